# Initial kernel scaffold; baseline (speedup 1.0000x reference)
import jax
import jax.numpy as jnp
from jax.experimental import pallas as pl


def kernel(x, n_id, batch_size, hist_emb):
    raise NotImplementedError("write your pallas kernel here")


# speedup vs baseline: 14.9934x; 14.9934x over previous
import functools

import jax
import jax.numpy as jnp
from jax import lax
from jax.experimental import pallas as pl
from jax.experimental.pallas import tpu as pltpu
from jax.experimental.pallas import tpu_sc as plsc

N_TOTAL = 16384
BS = 8192
HID = 256
NN = 100000
NC = 2
NS = 16
NW = NC * NS
L = 16

CHUNK = 6256
TAB = NS * CHUNK
RPT = BS // NW
RC = 128
NCH = RPT // RC
IN_GROUPS = BS // L
TAB_GROUPS = CHUNK // L


def _body(x_hbm, nid_hbm, hist_hbm, out_hbm,
          inids_v, tab_v, tmp_v, outids_v, pos_v, xidx_v, sidx_v, buf_a,
          buf_b, tab_sh, sem_a, sem_b, sem_s):
    c = lax.axis_index("c")
    s = lax.axis_index("s")
    wid = c * NS + s
    iota = lax.iota(jnp.int32, L)
    last_lane = iota == (L - 1)

    pltpu.sync_copy(nid_hbm.at[pl.ds(0, BS)], inids_v)

    neg1 = jnp.full((L,), -1, jnp.int32)

    def init_body(g, carry):
        tab_v[pl.ds(g * L, L)] = neg1
        return carry

    lax.fori_loop(0, TAB_GROUPS, init_body, 0)

    base = s * CHUNK

    def scan_body(g, carry):
        ids = inids_v[pl.ds(g * L, L)]
        li = ids - base
        inr = (li >= 0) & (li < CHUNK)
        lic = jnp.where(inr, li, CHUNK + iota)
        j16 = g * L + iota
        plsc.store_scatter(tmp_v, [lic], iota)
        dup = jnp.any(plsc.load_gather(tmp_v, [lic]) != iota)

        @pl.when(jnp.logical_not(dup))
        def _fast():
            plsc.store_scatter(tab_v, [jnp.where(inr, li, 0)], j16,
                               mask=inr)

        @pl.when(dup)
        def _slow():
            for k in range(L):
                plsc.store_scatter(tab_v, [jnp.where(inr, li, 0)], j16,
                                   mask=inr & (iota == k))

        return carry

    lax.fori_loop(0, IN_GROUPS, scan_body, 0)

    pltpu.sync_copy(tab_v, tab_sh.at[pl.ds(base, CHUNK)])
    plsc.subcore_barrier()

    row0 = wid * RPT
    for ch in range(NCH):
        pltpu.sync_copy(nid_hbm.at[pl.ds(BS + row0 + ch * RC, RC)],
                        outids_v.at[ch])
        pltpu.async_copy(tab_sh.at[outids_v.at[ch]], pos_v.at[ch],
                         sem_a).wait()
        for gg in range(RC // L):
            pos16 = pos_v[ch, pl.ds(gg * L, L)]
            q16 = row0 + ch * RC + gg * L + iota
            m = pos16 >= 0
            xidx_v[ch, pl.ds(gg * L, L)] = jnp.where(m, pos16, q16)
            sidx_v[ch, pl.ds(gg * L, L)] = jnp.where(m, BS + q16, q16)

    for ch in range(NCH):
        hist_dma = pltpu.async_copy(hist_hbm.at[outids_v.at[ch]], buf_a,
                                    sem_a)
        x_dma = pltpu.async_copy(x_hbm.at[xidx_v.at[ch]], buf_b, sem_b)
        hist_dma.wait()
        pltpu.sync_copy(buf_a, out_hbm.at[pl.ds(BS + row0 + ch * RC, RC)])
        x_dma.wait()
        pltpu.async_copy(buf_b, out_hbm.at[sidx_v.at[ch]], sem_s).wait()

    for ch in range(NCH):
        pltpu.sync_copy(x_hbm.at[pl.ds(row0 + ch * RC, RC)], buf_a)
        pltpu.sync_copy(buf_a, out_hbm.at[pl.ds(row0 + ch * RC, RC)])


@functools.partial(jax.jit, static_argnums=())
def kernel(x, n_id, batch_size, hist_emb):
    del batch_size
    mesh = plsc.VectorSubcoreMesh(core_axis_name="c", subcore_axis_name="s")
    run = pl.kernel(
        _body,
        out_type=jax.ShapeDtypeStruct((N_TOTAL, HID), jnp.float32),
        mesh=mesh,
        compiler_params=pltpu.CompilerParams(needs_layout_passes=False),
        scratch_types=[
            pltpu.VMEM((BS,), jnp.int32),
            pltpu.VMEM((CHUNK,), jnp.int32),
            pltpu.VMEM((CHUNK + L,), jnp.int32),
            pltpu.VMEM((NCH, RC), jnp.int32),
            pltpu.VMEM((NCH, RC), jnp.int32),
            pltpu.VMEM((NCH, RC), jnp.int32),
            pltpu.VMEM((NCH, RC), jnp.int32),
            pltpu.VMEM((RC, HID), jnp.float32),
            pltpu.VMEM((RC, HID), jnp.float32),
            pltpu.VMEM_SHARED((TAB,), jnp.int32),
            pltpu.SemaphoreType.DMA,
            pltpu.SemaphoreType.DMA,
            pltpu.SemaphoreType.DMA,
        ],
    )
    return run(x, n_id, hist_emb)


# speedup vs baseline: 15.2030x; 1.0140x over previous
import functools

import jax
import jax.numpy as jnp
from jax import lax
from jax.experimental import pallas as pl
from jax.experimental.pallas import tpu as pltpu
from jax.experimental.pallas import tpu_sc as plsc

N_TOTAL = 16384
BS = 8192
HID = 256
NN = 100000
NC = 2
NS = 16
NW = NC * NS
L = 16

CHUNK = 6256
TAB = NS * CHUNK
RPT = BS // NW
RC = 128
NCH = RPT // RC
IN_GROUPS = BS // L
TAB_GROUPS = CHUNK // L


def _body(x_hbm, nid_hbm, hist_hbm, out_hbm,
          inids_v, tab_v, tmp_v, outids_v, pos_v, xidx_v, sidx_v, buf_a,
          buf_b, tab_sh, sem_a, sem_b, sem_s):
    c = lax.axis_index("c")
    s = lax.axis_index("s")
    wid = c * NS + s
    iota = lax.iota(jnp.int32, L)
    last_lane = iota == (L - 1)

    pltpu.sync_copy(nid_hbm.at[pl.ds(0, BS)], inids_v)

    neg1 = jnp.full((L,), -1, jnp.int32)

    def init_body(g, carry):
        tab_v[pl.ds(g * L, L)] = neg1
        return carry

    lax.fori_loop(0, TAB_GROUPS, init_body, 0)

    base = s * CHUNK

    def scan_body(g, carry):
        ids = inids_v[pl.ds(g * L, L)]
        li = ids - base
        inr = (li >= 0) & (li < CHUNK)
        nin = jnp.sum(inr.astype(jnp.int32))

        @pl.when(nin > 0)
        def _any():
            j16 = g * L + iota
            lic = jnp.where(inr, li, 0)

            @pl.when(nin == 1)
            def _one():
                plsc.store_scatter(tab_v, [lic], j16, mask=inr)

            @pl.when(nin > 1)
            def _multi():
                lic2 = jnp.where(inr, li, CHUNK + iota)
                plsc.store_scatter(tmp_v, [lic2], j16)
                dup = jnp.any(plsc.load_gather(tmp_v, [lic2]) != j16)

                @pl.when(jnp.logical_not(dup))
                def _fast():
                    plsc.store_scatter(tab_v, [lic], j16, mask=inr)

                @pl.when(dup)
                def _slow():
                    for k in range(L):
                        plsc.store_scatter(tab_v, [lic], j16,
                                           mask=inr & (iota == k))

        return carry

    lax.fori_loop(0, IN_GROUPS, scan_body, 0)

    pltpu.sync_copy(tab_v, tab_sh.at[pl.ds(base, CHUNK)])
    plsc.subcore_barrier()

    row0 = wid * RPT
    for ch in range(NCH):
        pltpu.sync_copy(nid_hbm.at[pl.ds(BS + row0 + ch * RC, RC)],
                        outids_v.at[ch])
        pltpu.async_copy(tab_sh.at[outids_v.at[ch]], pos_v.at[ch],
                         sem_a).wait()
        for gg in range(RC // L):
            pos16 = pos_v[ch, pl.ds(gg * L, L)]
            q16 = row0 + ch * RC + gg * L + iota
            m = pos16 >= 0
            xidx_v[ch, pl.ds(gg * L, L)] = jnp.where(m, pos16, q16)
            sidx_v[ch, pl.ds(gg * L, L)] = jnp.where(m, BS + q16, q16)

    for ch in range(NCH):
        hist_dma = pltpu.async_copy(hist_hbm.at[outids_v.at[ch]], buf_a,
                                    sem_a)
        x_dma = pltpu.async_copy(x_hbm.at[xidx_v.at[ch]], buf_b, sem_b)
        hist_dma.wait()
        pltpu.sync_copy(buf_a, out_hbm.at[pl.ds(BS + row0 + ch * RC, RC)])
        x_dma.wait()
        pltpu.async_copy(buf_b, out_hbm.at[sidx_v.at[ch]], sem_s).wait()

    for ch in range(NCH):
        pltpu.sync_copy(x_hbm.at[pl.ds(row0 + ch * RC, RC)], buf_a)
        pltpu.sync_copy(buf_a, out_hbm.at[pl.ds(row0 + ch * RC, RC)])


@functools.partial(jax.jit, static_argnums=())
def kernel(x, n_id, batch_size, hist_emb):
    del batch_size
    mesh = plsc.VectorSubcoreMesh(core_axis_name="c", subcore_axis_name="s")
    run = pl.kernel(
        _body,
        out_type=jax.ShapeDtypeStruct((N_TOTAL, HID), jnp.float32),
        mesh=mesh,
        compiler_params=pltpu.CompilerParams(needs_layout_passes=False),
        scratch_types=[
            pltpu.VMEM((BS,), jnp.int32),
            pltpu.VMEM((CHUNK,), jnp.int32),
            pltpu.VMEM((CHUNK + L,), jnp.int32),
            pltpu.VMEM((NCH, RC), jnp.int32),
            pltpu.VMEM((NCH, RC), jnp.int32),
            pltpu.VMEM((NCH, RC), jnp.int32),
            pltpu.VMEM((NCH, RC), jnp.int32),
            pltpu.VMEM((RC, HID), jnp.float32),
            pltpu.VMEM((RC, HID), jnp.float32),
            pltpu.VMEM_SHARED((TAB,), jnp.int32),
            pltpu.SemaphoreType.DMA,
            pltpu.SemaphoreType.DMA,
            pltpu.SemaphoreType.DMA,
        ],
    )
    return run(x, n_id, hist_emb)
